# baseline (speedup 1.0000x reference)
import dataclasses
import functools

import jax
import jax.numpy as jnp
from jax import lax
from jax.experimental import pallas as pl
from jax.experimental.pallas import tpu as pltpu
from jax.experimental.pallas import tpu_sc as plsc

F_CH = 64
B_CH = 128
BH = 128
BW = 128
ND = 32
D_MIN = 1.0
D_MAX = 50.0
EXT = 60.0
NB = 2
NV = 6
HF = 32
WF = 88
NPIX = HF * WF
BV = NB * NV
TRASH = BH * BW
ACC_ROWS = TRASH + 128
NCH = NPIX // 128
N_ITEMS = NV * NCH
PREC = lax.Precision.HIGHEST


def _gelu(x):
    return 0.5 * x * (1.0 + lax.erf(x * (1.0 / jnp.sqrt(2.0).astype(jnp.float32))))


def _dot4(x, w):
    return lax.dot_general(x, w, (((x.ndim - 1,), (0,)), ((), ())),
                           precision=PREC, preferred_element_type=jnp.float32)


def _bn(h, g, be, axes):
    m = jnp.mean(h, axis=axes, keepdims=True)
    v = jnp.mean((h - m) ** 2, axis=axes, keepdims=True)
    return (h - m) / jnp.sqrt(v + 1e-5) * g + be


def _cam1_kernel(xpad_ref, w9_ref, b1_ref, fpw_ref, fpb_ref,
                 c1_ref, fpr_ref, st1_ref, stf_ref, s1_acc, sf_acc):
    bv = pl.program_id(0)
    x = xpad_ref[0, 1:HF + 1, 1:WF + 1, :]
    acc = jnp.broadcast_to(b1_ref[0], (HF, WF, F_CH))
    for ky in range(3):
        for kx in range(3):
            sl = xpad_ref[0, ky:ky + HF, kx:kx + WF, :]
            acc = acc + _dot4(sl, w9_ref[ky * 3 + kx])
    c1_ref[0] = acc
    fpr = (_dot4(x, fpw_ref[...]) + fpb_ref[0]).reshape(NPIX, B_CH)
    fpr_ref[0] = fpr

    @pl.when(bv == 0)
    def _():
        s1_acc[...] = jnp.zeros_like(s1_acc)
        sf_acc[...] = jnp.zeros_like(sf_acc)

    a2 = acc.reshape(NPIX, F_CH)
    s1_acc[0:1, :] += jnp.sum(a2, axis=0, keepdims=True)
    s1_acc[1:2, :] += jnp.sum(a2 * a2, axis=0, keepdims=True)
    sf_acc[0:1, :] += jnp.sum(fpr, axis=0, keepdims=True)
    sf_acc[1:2, :] += jnp.sum(fpr * fpr, axis=0, keepdims=True)

    @pl.when(bv == BV - 1)
    def _():
        st1_ref[...] = s1_acc[...]
        stf_ref[...] = sf_acc[...]


def _cam2_kernel(c1_ref, fpr_ref, st1_ref, stf_ref, g1_ref, be1_ref,
                 w2_ref, b2_ref, fpg_ref, fpbe_ref, vt_ref, fp_ref, w_ref):
    cnt = float(BV * NPIX)
    m1 = st1_ref[0, :] / cnt
    v1 = st1_ref[1, :] / cnt - m1 * m1
    h = (c1_ref[0] - m1) / jnp.sqrt(v1 + 1e-5) * g1_ref[0] + be1_ref[0]
    h = _gelu(h)
    logits = _dot4(h, w2_ref[...]) + b2_ref[0]
    dp = jax.nn.softmax(logits, axis=-1)
    w_ref[0] = dp.reshape(NPIX, ND) * vt_ref[0]
    mf = stf_ref[0, :] / cnt
    vf = stf_ref[1, :] / cnt - mf * mf
    fp = (fpr_ref[0] - mf) / jnp.sqrt(vf + 1e-5) * fpg_ref[0] + fpbe_ref[0]
    fp_ref[0] = _gelu(fp)


def _cam_call(xpad, w9, b1, g1, be1, w2, b2, fpw, fpb, fpg, fpbe, vt,
              interpret=False):
    c1, fpr, st1, stf = pl.pallas_call(
        _cam1_kernel,
        grid=(BV,),
        in_specs=[
            pl.BlockSpec((1, HF + 2, WF + 2, F_CH), lambda i: (i, 0, 0, 0)),
            pl.BlockSpec((9, F_CH, F_CH), lambda i: (0, 0, 0)),
            pl.BlockSpec((1, F_CH), lambda i: (0, 0)),
            pl.BlockSpec((F_CH, B_CH), lambda i: (0, 0)),
            pl.BlockSpec((1, B_CH), lambda i: (0, 0)),
        ],
        out_specs=[
            pl.BlockSpec((1, HF, WF, F_CH), lambda i: (i, 0, 0, 0)),
            pl.BlockSpec((1, NPIX, B_CH), lambda i: (i, 0, 0)),
            pl.BlockSpec((2, F_CH), lambda i: (0, 0)),
            pl.BlockSpec((2, B_CH), lambda i: (0, 0)),
        ],
        out_shape=(
            jax.ShapeDtypeStruct((BV, HF, WF, F_CH), jnp.float32),
            jax.ShapeDtypeStruct((BV, NPIX, B_CH), jnp.float32),
            jax.ShapeDtypeStruct((2, F_CH), jnp.float32),
            jax.ShapeDtypeStruct((2, B_CH), jnp.float32),
        ),
        scratch_shapes=[
            pltpu.VMEM((2, F_CH), jnp.float32),
            pltpu.VMEM((2, B_CH), jnp.float32),
        ],
        interpret=interpret,
    )(xpad, w9, b1, fpw, fpb)

    fp, w_arr = pl.pallas_call(
        _cam2_kernel,
        grid=(BV,),
        in_specs=[
            pl.BlockSpec((1, HF, WF, F_CH), lambda i: (i, 0, 0, 0)),
            pl.BlockSpec((1, NPIX, B_CH), lambda i: (i, 0, 0)),
            pl.BlockSpec((2, F_CH), lambda i: (0, 0)),
            pl.BlockSpec((2, B_CH), lambda i: (0, 0)),
            pl.BlockSpec((1, F_CH), lambda i: (0, 0)),
            pl.BlockSpec((1, F_CH), lambda i: (0, 0)),
            pl.BlockSpec((F_CH, ND), lambda i: (0, 0)),
            pl.BlockSpec((1, ND), lambda i: (0, 0)),
            pl.BlockSpec((1, B_CH), lambda i: (0, 0)),
            pl.BlockSpec((1, B_CH), lambda i: (0, 0)),
            pl.BlockSpec((1, NPIX, ND), lambda i: (i, 0, 0)),
        ],
        out_specs=[
            pl.BlockSpec((1, NPIX, B_CH), lambda i: (i, 0, 0)),
            pl.BlockSpec((1, NPIX, ND), lambda i: (i, 0, 0)),
        ],
        out_shape=(
            jax.ShapeDtypeStruct((BV, NPIX, B_CH), jnp.float32),
            jax.ShapeDtypeStruct((BV, NPIX, ND), jnp.float32),
        ),
        interpret=interpret,
    )(c1, fpr, st1, stf, g1, be1, w2, b2, fpg, fpbe, vt)
    return fp, w_arr


ACC_C = TRASH + 16
DB = 4


def _splat_call(fpT, wr, idxr):
    mesh = plsc.VectorSubcoreMesh(core_axis_name="c", subcore_axis_name="s")
    cp = pltpu.CompilerParams()
    if "needs_layout_passes" in pltpu.CompilerParams.__dataclass_fields__:
        cp = dataclasses.replace(cp, needs_layout_passes=False)

    @functools.partial(
        pl.kernel,
        mesh=mesh,
        compiler_params=cp,
        out_type=jax.ShapeDtypeStruct((NB, B_CH, TRASH), jnp.float32),
        scratch_types=[
            pltpu.VMEM((DB, NPIX), jnp.int32),
            pltpu.VMEM((DB, NPIX), jnp.float32),
            pltpu.VMEM((4, NPIX), jnp.float32),
            pltpu.VMEM((4, ACC_C), jnp.float32),
        ],
    )
    def sc_kernel(fpT_hbm, wr_hbm, idxr_hbm, out_hbm, idx_v, w_v, fp_v, acc):
        cid = lax.axis_index("c")
        sid = lax.axis_index("s")
        for hp in range(2):
            c0 = hp * 64 + sid * 4

            @pl.loop(0, ACC_C // 16)
            def _(t):
                z = jnp.zeros((16,), jnp.float32)
                for c in range(4):
                    acc[c, pl.ds(t * 16, 16)] = z

            for v in range(NV):
                bv = cid * NV + v
                pltpu.sync_copy(fpT_hbm.at[bv, pl.ds(c0, 4)], fp_v)

                @pl.loop(0, ND // DB)
                def _(db):
                    pltpu.sync_copy(idxr_hbm.at[bv, pl.ds(db * DB, DB)],
                                    idx_v)
                    pltpu.sync_copy(wr_hbm.at[bv, pl.ds(db * DB, DB)], w_v)

                    @pl.loop(0, DB)
                    def _(d):
                        @pl.loop(0, NPIX // 16)
                        def _(g):
                            s = pl.ds(g * 16, 16)
                            ivec = idx_v[d, s]
                            wvec = w_v[d, s]
                            for c in range(4):
                                contrib = fp_v[c, s] * wvec
                                plsc.addupdate_scatter(
                                    acc,
                                    [jnp.full((16,), c, jnp.int32), ivec],
                                    contrib)

            for c in range(4):
                pltpu.sync_copy(acc.at[c, pl.ds(0, TRASH)],
                                out_hbm.at[cid, c0 + c])

    return sc_kernel(fpT, wr, idxr)


BAND = 8
NBANDS = BH // BAND


def _rconv_kernel(xpad_ref, w9_ref, b_ref, out_ref, st_ref, s_acc):
    b = pl.program_id(0)
    r = pl.program_id(1)
    acc = jnp.broadcast_to(b_ref[0], (BAND, BW, B_CH))
    for ky in range(3):
        for kx in range(3):
            sl = xpad_ref[0, pl.ds(r * BAND + ky, BAND), kx:kx + BW, :]
            acc = acc + _dot4(sl, w9_ref[ky * 3 + kx])
    out_ref[0] = acc

    @pl.when((b == 0) & (r == 0))
    def _():
        s_acc[...] = jnp.zeros_like(s_acc)

    a2 = acc.reshape(BAND * BW, B_CH)
    s_acc[0:1, :] += jnp.sum(a2, axis=0, keepdims=True)
    s_acc[1:2, :] += jnp.sum(a2 * a2, axis=0, keepdims=True)

    @pl.when((b == NB - 1) & (r == NBANDS - 1))
    def _():
        st_ref[...] = s_acc[...]


def _rnorm_kernel(c_ref, st_ref, g_ref, be_ref, out_ref):
    cnt = float(NB * BH * BW)
    m = st_ref[0, :] / cnt
    v = st_ref[1, :] / cnt - m * m
    h = (c_ref[0] - m) / jnp.sqrt(v + 1e-5) * g_ref[0] + be_ref[0]
    out_ref[0] = _gelu(h)


def _refine_call(xpad, w9, b, g, be, interpret=False):
    conv, st = pl.pallas_call(
        _rconv_kernel,
        grid=(NB, NBANDS),
        in_specs=[
            pl.BlockSpec((1, BH + 2, BW + 2, B_CH), lambda b, r: (b, 0, 0, 0)),
            pl.BlockSpec((9, B_CH, B_CH), lambda b, r: (0, 0, 0)),
            pl.BlockSpec((1, B_CH), lambda b, r: (0, 0)),
        ],
        out_specs=[
            pl.BlockSpec((1, BAND, BW, B_CH), lambda b, r: (b, r, 0, 0)),
            pl.BlockSpec((2, B_CH), lambda b, r: (0, 0)),
        ],
        out_shape=(
            jax.ShapeDtypeStruct((NB, BH, BW, B_CH), jnp.float32),
            jax.ShapeDtypeStruct((2, B_CH), jnp.float32),
        ),
        scratch_shapes=[pltpu.VMEM((2, B_CH), jnp.float32)],
        interpret=interpret,
    )(xpad, w9, b)

    return pl.pallas_call(
        _rnorm_kernel,
        grid=(NB, NBANDS),
        in_specs=[
            pl.BlockSpec((1, BAND, BW, B_CH), lambda b, r: (b, r, 0, 0)),
            pl.BlockSpec((2, B_CH), lambda b, r: (0, 0)),
            pl.BlockSpec((1, B_CH), lambda b, r: (0, 0)),
            pl.BlockSpec((1, B_CH), lambda b, r: (0, 0)),
        ],
        out_specs=pl.BlockSpec((1, BAND, BW, B_CH), lambda b, r: (b, r, 0, 0)),
        out_shape=jax.ShapeDtypeStruct((NB, BH, BW, B_CH), jnp.float32),
        interpret=interpret,
    )(conv, st, g, be)


def kernel(feat_maps, K_list, T_list, trust,
           dh_w1, dh_b1, dh_g1, dh_be1, dh_w2, dh_b2,
           fp_w, fp_b, fp_g, fp_be,
           br_w1, br_b1, br_g1, br_be1, br_w2, br_b2, br_g2, br_be2):
    x = feat_maps.reshape(BV, F_CH, HF, WF).transpose(0, 2, 3, 1)
    xpad = jnp.pad(x, ((0, 0), (1, 1), (1, 1), (0, 0)))
    w9 = dh_w1.transpose(2, 3, 1, 0).reshape(9, F_CH, F_CH)
    w2m = dh_w2.reshape(ND, F_CH).T
    fpwm = fp_w.reshape(B_CH, F_CH).T

    ys = jnp.arange(HF, dtype=jnp.float32)
    xs = jnp.arange(WF, dtype=jnp.float32)
    yy, xx = jnp.meshgrid(ys, xs, indexing='ij')
    pixh = jnp.stack([xx.ravel(), yy.ravel(), jnp.ones(NPIX, jnp.float32)],
                     axis=0)
    depths = jnp.linspace(D_MIN, D_MAX, ND)
    idx_list, vt_list = [], []
    for b in range(NB):
        for v in range(NV):
            Kinv = jnp.linalg.inv(K_list[b, v])
            rays = Kinv @ pixh
            pts_cam = jnp.transpose(rays[:, :, None] * depths[None, None, :],
                                    (1, 2, 0)).reshape(-1, 3)
            T = T_list[b, v]
            pts_ego = pts_cam @ T[:3, :3].T + T[:3, 3][None, :]
            u = (pts_ego[:, 0] + EXT) / (2.0 * EXT)
            w_ = (pts_ego[:, 1] + EXT) / (2.0 * EXT)
            col = jnp.clip(u * (BW - 1), -2.0 ** 30, 2.0 ** 30).astype(jnp.int32)
            row = jnp.clip(w_ * (BH - 1), -2.0 ** 30, 2.0 ** 30).astype(jnp.int32)
            valid = (col >= 0) & (col < BW) & (row >= 0) & (row < BH)
            idx_list.append(jnp.where(valid, row * BW + col, TRASH))
            vt_list.append(valid.astype(jnp.float32) * trust[b, v])
    idx_arr = jnp.stack(idx_list).reshape(BV, NPIX, ND)
    vt = jnp.stack(vt_list).reshape(BV, NPIX, ND)

    fp, w_arr = _cam_call(xpad, w9, dh_b1.reshape(1, F_CH),
                          dh_g1.reshape(1, F_CH), dh_be1.reshape(1, F_CH),
                          w2m, dh_b2.reshape(1, ND), fpwm,
                          fp_b.reshape(1, B_CH), fp_g.reshape(1, B_CH),
                          fp_be.reshape(1, B_CH), vt)

    idxr = idx_arr.transpose(0, 2, 1)
    wr = w_arr.transpose(0, 2, 1)
    fpT = fp.transpose(0, 2, 1)

    bev_t = _splat_call(fpT, wr, idxr)
    bev = bev_t.transpose(0, 2, 1).reshape(NB, BH, BW, B_CH)

    brw1 = br_w1.transpose(2, 3, 1, 0).reshape(9, B_CH, B_CH)
    brw2 = br_w2.transpose(2, 3, 1, 0).reshape(9, B_CH, B_CH)
    x1 = _refine_call(jnp.pad(bev, ((0, 0), (1, 1), (1, 1), (0, 0))),
                      brw1, br_b1.reshape(1, B_CH), br_g1.reshape(1, B_CH),
                      br_be1.reshape(1, B_CH))
    x2 = _refine_call(jnp.pad(x1, ((0, 0), (1, 1), (1, 1), (0, 0))),
                      brw2, br_b2.reshape(1, B_CH), br_g2.reshape(1, B_CH),
                      br_be2.reshape(1, B_CH))
    return x2.transpose(0, 3, 1, 2)


# speedup vs baseline: 1.7715x; 1.7715x over previous
import dataclasses
import functools

import jax
import jax.numpy as jnp
from jax import lax
from jax.experimental import pallas as pl
from jax.experimental.pallas import tpu as pltpu
from jax.experimental.pallas import tpu_sc as plsc

F_CH = 64
B_CH = 128
BH = 128
BW = 128
ND = 32
D_MIN = 1.0
D_MAX = 50.0
EXT = 60.0
NB = 2
NV = 6
HF = 32
WF = 88
NPIX = HF * WF
BV = NB * NV
TRASH = BH * BW
ACC_ROWS = TRASH + 128
NCH = NPIX // 128
N_ITEMS = NV * NCH
PREC = lax.Precision.HIGHEST


def _gelu(x):
    return 0.5 * x * (1.0 + lax.erf(x * (1.0 / jnp.sqrt(2.0).astype(jnp.float32))))


def _dot4(x, w):
    return lax.dot_general(x, w, (((x.ndim - 1,), (0,)), ((), ())),
                           precision=PREC, preferred_element_type=jnp.float32)


def _bn(h, g, be, axes):
    m = jnp.mean(h, axis=axes, keepdims=True)
    v = jnp.mean((h - m) ** 2, axis=axes, keepdims=True)
    return (h - m) / jnp.sqrt(v + 1e-5) * g + be


def _cam1_kernel(xpad_ref, w9_ref, b1_ref, fpw_ref, fpb_ref,
                 c1_ref, fpr_ref, st1_ref, stf_ref, s1_acc, sf_acc):
    bv = pl.program_id(0)
    x = xpad_ref[0, 1:HF + 1, 1:WF + 1, :]
    acc = jnp.broadcast_to(b1_ref[0], (HF, WF, F_CH))
    for ky in range(3):
        for kx in range(3):
            sl = xpad_ref[0, ky:ky + HF, kx:kx + WF, :]
            acc = acc + _dot4(sl, w9_ref[ky * 3 + kx])
    c1_ref[0] = acc
    fpr = (_dot4(x, fpw_ref[...]) + fpb_ref[0]).reshape(NPIX, B_CH)
    fpr_ref[0] = fpr

    @pl.when(bv == 0)
    def _():
        s1_acc[...] = jnp.zeros_like(s1_acc)
        sf_acc[...] = jnp.zeros_like(sf_acc)

    a2 = acc.reshape(NPIX, F_CH)
    s1_acc[0:1, :] += jnp.sum(a2, axis=0, keepdims=True)
    s1_acc[1:2, :] += jnp.sum(a2 * a2, axis=0, keepdims=True)
    sf_acc[0:1, :] += jnp.sum(fpr, axis=0, keepdims=True)
    sf_acc[1:2, :] += jnp.sum(fpr * fpr, axis=0, keepdims=True)

    @pl.when(bv == BV - 1)
    def _():
        st1_ref[...] = s1_acc[...]
        stf_ref[...] = sf_acc[...]


def _cam2_kernel(c1_ref, fpr_ref, st1_ref, stf_ref, g1_ref, be1_ref,
                 w2_ref, b2_ref, fpg_ref, fpbe_ref, vt_ref, fp_ref, w_ref):
    cnt = float(BV * NPIX)
    m1 = st1_ref[0, :] / cnt
    v1 = st1_ref[1, :] / cnt - m1 * m1
    h = (c1_ref[0] - m1) / jnp.sqrt(v1 + 1e-5) * g1_ref[0] + be1_ref[0]
    h = _gelu(h)
    logits = _dot4(h, w2_ref[...]) + b2_ref[0]
    dp = jax.nn.softmax(logits, axis=-1)
    w_ref[0] = dp.reshape(NPIX, ND) * vt_ref[0]
    mf = stf_ref[0, :] / cnt
    vf = stf_ref[1, :] / cnt - mf * mf
    fp = (fpr_ref[0] - mf) / jnp.sqrt(vf + 1e-5) * fpg_ref[0] + fpbe_ref[0]
    fp_ref[0] = _gelu(fp)


def _cam_call(xpad, w9, b1, g1, be1, w2, b2, fpw, fpb, fpg, fpbe, vt,
              interpret=False):
    c1, fpr, st1, stf = pl.pallas_call(
        _cam1_kernel,
        grid=(BV,),
        in_specs=[
            pl.BlockSpec((1, HF + 2, WF + 2, F_CH), lambda i: (i, 0, 0, 0)),
            pl.BlockSpec((9, F_CH, F_CH), lambda i: (0, 0, 0)),
            pl.BlockSpec((1, F_CH), lambda i: (0, 0)),
            pl.BlockSpec((F_CH, B_CH), lambda i: (0, 0)),
            pl.BlockSpec((1, B_CH), lambda i: (0, 0)),
        ],
        out_specs=[
            pl.BlockSpec((1, HF, WF, F_CH), lambda i: (i, 0, 0, 0)),
            pl.BlockSpec((1, NPIX, B_CH), lambda i: (i, 0, 0)),
            pl.BlockSpec((2, F_CH), lambda i: (0, 0)),
            pl.BlockSpec((2, B_CH), lambda i: (0, 0)),
        ],
        out_shape=(
            jax.ShapeDtypeStruct((BV, HF, WF, F_CH), jnp.float32),
            jax.ShapeDtypeStruct((BV, NPIX, B_CH), jnp.float32),
            jax.ShapeDtypeStruct((2, F_CH), jnp.float32),
            jax.ShapeDtypeStruct((2, B_CH), jnp.float32),
        ),
        scratch_shapes=[
            pltpu.VMEM((2, F_CH), jnp.float32),
            pltpu.VMEM((2, B_CH), jnp.float32),
        ],
        interpret=interpret,
    )(xpad, w9, b1, fpw, fpb)

    fp, w_arr = pl.pallas_call(
        _cam2_kernel,
        grid=(BV,),
        in_specs=[
            pl.BlockSpec((1, HF, WF, F_CH), lambda i: (i, 0, 0, 0)),
            pl.BlockSpec((1, NPIX, B_CH), lambda i: (i, 0, 0)),
            pl.BlockSpec((2, F_CH), lambda i: (0, 0)),
            pl.BlockSpec((2, B_CH), lambda i: (0, 0)),
            pl.BlockSpec((1, F_CH), lambda i: (0, 0)),
            pl.BlockSpec((1, F_CH), lambda i: (0, 0)),
            pl.BlockSpec((F_CH, ND), lambda i: (0, 0)),
            pl.BlockSpec((1, ND), lambda i: (0, 0)),
            pl.BlockSpec((1, B_CH), lambda i: (0, 0)),
            pl.BlockSpec((1, B_CH), lambda i: (0, 0)),
            pl.BlockSpec((1, NPIX, ND), lambda i: (i, 0, 0)),
        ],
        out_specs=[
            pl.BlockSpec((1, NPIX, B_CH), lambda i: (i, 0, 0)),
            pl.BlockSpec((1, NPIX, ND), lambda i: (i, 0, 0)),
        ],
        out_shape=(
            jax.ShapeDtypeStruct((BV, NPIX, B_CH), jnp.float32),
            jax.ShapeDtypeStruct((BV, NPIX, ND), jnp.float32),
        ),
        interpret=interpret,
    )(c1, fpr, st1, stf, g1, be1, w2, b2, fpg, fpbe, vt)
    return fp, w_arr


ACC_C = TRASH + 16
DB = 4


def _splat_call(fpT, wr, idxr):
    mesh = plsc.VectorSubcoreMesh(core_axis_name="c", subcore_axis_name="s")
    cp = pltpu.CompilerParams()
    if "needs_layout_passes" in pltpu.CompilerParams.__dataclass_fields__:
        cp = dataclasses.replace(cp, needs_layout_passes=False)

    @functools.partial(
        pl.kernel,
        mesh=mesh,
        compiler_params=cp,
        out_type=jax.ShapeDtypeStruct((NB, B_CH, TRASH), jnp.float32),
        scratch_types=[
            pltpu.VMEM((DB, NPIX), jnp.int32),
            pltpu.VMEM((DB, NPIX), jnp.float32),
            pltpu.VMEM((4, NPIX), jnp.float32),
            pltpu.VMEM((4, ACC_C), jnp.float32),
        ],
    )
    def sc_kernel(fpT_hbm, wr_hbm, idxr_hbm, out_hbm, idx_v, w_v, fp_v, acc):
        cid = lax.axis_index("c")
        sid = lax.axis_index("s")
        for hp in range(2):
            c0 = hp * 64 + sid * 4

            @pl.loop(0, ACC_C // 16)
            def _(t):
                z = jnp.zeros((16,), jnp.float32)
                for c in range(4):
                    acc[c, pl.ds(t * 16, 16)] = z

            for v in range(NV):
                bv = cid * NV + v
                pltpu.sync_copy(fpT_hbm.at[bv, pl.ds(c0, 4)], fp_v)

                @pl.loop(0, ND // DB)
                def _(db):
                    pltpu.sync_copy(idxr_hbm.at[bv, pl.ds(db * DB, DB)],
                                    idx_v)
                    pltpu.sync_copy(wr_hbm.at[bv, pl.ds(db * DB, DB)], w_v)

                    @pl.loop(0, DB)
                    def _(d):
                        @pl.loop(0, NPIX // 16)
                        def _(g):
                            s = pl.ds(g * 16, 16)
                            ivec = idx_v[d, s]

                            @pl.when(jnp.min(ivec) < TRASH)
                            def _():
                                wvec = w_v[d, s]
                                for c in range(4):
                                    contrib = fp_v[c, s] * wvec
                                    plsc.addupdate_scatter(
                                        acc,
                                        [jnp.full((16,), c, jnp.int32), ivec],
                                        contrib)

            for c in range(4):
                pltpu.sync_copy(acc.at[c, pl.ds(0, TRASH)],
                                out_hbm.at[cid, c0 + c])

    return sc_kernel(fpT, wr, idxr)


BAND = 8
NBANDS = BH // BAND


def _rconv_kernel(xpad_ref, w9_ref, b_ref, out_ref, st_ref, s_acc):
    b = pl.program_id(0)
    r = pl.program_id(1)
    acc = jnp.broadcast_to(b_ref[0], (BAND, BW, B_CH))
    for ky in range(3):
        for kx in range(3):
            sl = xpad_ref[0, pl.ds(r * BAND + ky, BAND), kx:kx + BW, :]
            acc = acc + _dot4(sl, w9_ref[ky * 3 + kx])
    out_ref[0] = acc

    @pl.when((b == 0) & (r == 0))
    def _():
        s_acc[...] = jnp.zeros_like(s_acc)

    a2 = acc.reshape(BAND * BW, B_CH)
    s_acc[0:1, :] += jnp.sum(a2, axis=0, keepdims=True)
    s_acc[1:2, :] += jnp.sum(a2 * a2, axis=0, keepdims=True)

    @pl.when((b == NB - 1) & (r == NBANDS - 1))
    def _():
        st_ref[...] = s_acc[...]


def _rnorm_kernel(c_ref, st_ref, g_ref, be_ref, out_ref):
    cnt = float(NB * BH * BW)
    m = st_ref[0, :] / cnt
    v = st_ref[1, :] / cnt - m * m
    h = (c_ref[0] - m) / jnp.sqrt(v + 1e-5) * g_ref[0] + be_ref[0]
    out_ref[0] = _gelu(h)


def _refine_call(xpad, w9, b, g, be, interpret=False):
    conv, st = pl.pallas_call(
        _rconv_kernel,
        grid=(NB, NBANDS),
        in_specs=[
            pl.BlockSpec((1, BH + 2, BW + 2, B_CH), lambda b, r: (b, 0, 0, 0)),
            pl.BlockSpec((9, B_CH, B_CH), lambda b, r: (0, 0, 0)),
            pl.BlockSpec((1, B_CH), lambda b, r: (0, 0)),
        ],
        out_specs=[
            pl.BlockSpec((1, BAND, BW, B_CH), lambda b, r: (b, r, 0, 0)),
            pl.BlockSpec((2, B_CH), lambda b, r: (0, 0)),
        ],
        out_shape=(
            jax.ShapeDtypeStruct((NB, BH, BW, B_CH), jnp.float32),
            jax.ShapeDtypeStruct((2, B_CH), jnp.float32),
        ),
        scratch_shapes=[pltpu.VMEM((2, B_CH), jnp.float32)],
        interpret=interpret,
    )(xpad, w9, b)

    return pl.pallas_call(
        _rnorm_kernel,
        grid=(NB, NBANDS),
        in_specs=[
            pl.BlockSpec((1, BAND, BW, B_CH), lambda b, r: (b, r, 0, 0)),
            pl.BlockSpec((2, B_CH), lambda b, r: (0, 0)),
            pl.BlockSpec((1, B_CH), lambda b, r: (0, 0)),
            pl.BlockSpec((1, B_CH), lambda b, r: (0, 0)),
        ],
        out_specs=pl.BlockSpec((1, BAND, BW, B_CH), lambda b, r: (b, r, 0, 0)),
        out_shape=jax.ShapeDtypeStruct((NB, BH, BW, B_CH), jnp.float32),
        interpret=interpret,
    )(conv, st, g, be)


def kernel(feat_maps, K_list, T_list, trust,
           dh_w1, dh_b1, dh_g1, dh_be1, dh_w2, dh_b2,
           fp_w, fp_b, fp_g, fp_be,
           br_w1, br_b1, br_g1, br_be1, br_w2, br_b2, br_g2, br_be2):
    x = feat_maps.reshape(BV, F_CH, HF, WF).transpose(0, 2, 3, 1)
    xpad = jnp.pad(x, ((0, 0), (1, 1), (1, 1), (0, 0)))
    w9 = dh_w1.transpose(2, 3, 1, 0).reshape(9, F_CH, F_CH)
    w2m = dh_w2.reshape(ND, F_CH).T
    fpwm = fp_w.reshape(B_CH, F_CH).T

    ys = jnp.arange(HF, dtype=jnp.float32)
    xs = jnp.arange(WF, dtype=jnp.float32)
    yy, xx = jnp.meshgrid(ys, xs, indexing='ij')
    pixh = jnp.stack([xx.ravel(), yy.ravel(), jnp.ones(NPIX, jnp.float32)],
                     axis=0)
    depths = jnp.linspace(D_MIN, D_MAX, ND)
    idx_list, vt_list = [], []
    for b in range(NB):
        for v in range(NV):
            Kinv = jnp.linalg.inv(K_list[b, v])
            rays = Kinv @ pixh
            pts_cam = jnp.transpose(rays[:, :, None] * depths[None, None, :],
                                    (1, 2, 0)).reshape(-1, 3)
            T = T_list[b, v]
            pts_ego = pts_cam @ T[:3, :3].T + T[:3, 3][None, :]
            u = (pts_ego[:, 0] + EXT) / (2.0 * EXT)
            w_ = (pts_ego[:, 1] + EXT) / (2.0 * EXT)
            col = jnp.clip(u * (BW - 1), -2.0 ** 30, 2.0 ** 30).astype(jnp.int32)
            row = jnp.clip(w_ * (BH - 1), -2.0 ** 30, 2.0 ** 30).astype(jnp.int32)
            valid = (col >= 0) & (col < BW) & (row >= 0) & (row < BH)
            idx_list.append(jnp.where(valid, row * BW + col, TRASH))
            vt_list.append(valid.astype(jnp.float32) * trust[b, v])
    idx_arr = jnp.stack(idx_list).reshape(BV, NPIX, ND)
    vt = jnp.stack(vt_list).reshape(BV, NPIX, ND)

    fp, w_arr = _cam_call(xpad, w9, dh_b1.reshape(1, F_CH),
                          dh_g1.reshape(1, F_CH), dh_be1.reshape(1, F_CH),
                          w2m, dh_b2.reshape(1, ND), fpwm,
                          fp_b.reshape(1, B_CH), fp_g.reshape(1, B_CH),
                          fp_be.reshape(1, B_CH), vt)

    idxr = idx_arr.transpose(0, 2, 1)
    wr = w_arr.transpose(0, 2, 1)
    fpT = fp.transpose(0, 2, 1)

    bev_t = _splat_call(fpT, wr, idxr)
    bev = bev_t.transpose(0, 2, 1).reshape(NB, BH, BW, B_CH)

    brw1 = br_w1.transpose(2, 3, 1, 0).reshape(9, B_CH, B_CH)
    brw2 = br_w2.transpose(2, 3, 1, 0).reshape(9, B_CH, B_CH)
    x1 = _refine_call(jnp.pad(bev, ((0, 0), (1, 1), (1, 1), (0, 0))),
                      brw1, br_b1.reshape(1, B_CH), br_g1.reshape(1, B_CH),
                      br_be1.reshape(1, B_CH))
    x2 = _refine_call(jnp.pad(x1, ((0, 0), (1, 1), (1, 1), (0, 0))),
                      brw2, br_b2.reshape(1, B_CH), br_g2.reshape(1, B_CH),
                      br_be2.reshape(1, B_CH))
    return x2.transpose(0, 3, 1, 2)
